# baseline (speedup 1.0000x reference)
import jax
import jax.numpy as jnp
from jax import lax
from jax.experimental import pallas as pl
from jax.experimental.pallas import tpu as pltpu
from jax.experimental.pallas import tpu_sc as plsc

NC, NS = 2, 16
NW = NC * NS
BT = 128
NBUF = 2


def _sc_body(idx, table, out, idx_v, gbuf0, gbuf1, tbuf0, tbuf1,
             gsem0, gsem1, ssem0, ssem1):
    wid = lax.axis_index("s") * NC + lax.axis_index("c")
    L, B = idx.shape
    E = table.shape[1]
    b0 = wid * BT
    gbufs = (gbuf0, gbuf1)
    tbufs = (tbuf0, tbuf1)
    gsems = (gsem0, gsem1)
    ssems = (ssem0, ssem1)

    pltpu.sync_copy(idx.at[:, pl.ds(b0, BT)], idx_v)

    row16 = [jnp.arange(16, dtype=jnp.int32) + 16 * k for k in range(BT // 16)]

    def fire_gather(l, b):
        pltpu.async_copy(table.at[idx_v.at[l]], gbufs[b], gsems[b])

    def wait_gather(b):
        pltpu.make_async_copy(table.at[pl.ds(0, BT)], gbufs[b],
                              gsems[b]).wait()

    def wait_stores(b):
        for _ in range(E // 8):
            pltpu.make_async_copy(tbufs[b].at[pl.ds(0, 8)], out.at[0, 0, 0],
                                  ssems[b]).wait()

    def transpose_and_store(l, b):
        gb, tb = gbufs[b], tbufs[b]
        for e in range(E):
            col = jnp.full((16,), e, dtype=jnp.int32)
            for k in range(BT // 16):
                tb[e, pl.ds(16 * k, 16)] = plsc.load_gather(
                    gb, [row16[k], col])
        for et in range(E // 8):
            pltpu.async_copy(tb.at[pl.ds(8 * et, 8)], out.at[l, et, wid],
                             ssems[b])

    for b in range(NBUF):
        fire_gather(b, b)

    def body(i, carry):
        for b in range(NBUF):
            l = i * NBUF + b
            wait_gather(b)

            @pl.when(l >= NBUF)
            def _():
                wait_stores(b)

            transpose_and_store(l, b)

            @pl.when(l + NBUF < L)
            def _():
                fire_gather(l + NBUF, b)

        return carry

    lax.fori_loop(0, L // NBUF, body, 0)
    for b in range(NBUF):
        wait_stores(b)


def kernel(old_token_tensor, new_token_tensor, action_tensor, nl_tensor,
           code_table, action_table, nl_table):
    B, L = old_token_tensor.shape
    E = code_table.shape[1]
    mesh = plsc.VectorSubcoreMesh(
        core_axis_name="c", subcore_axis_name="s",
        num_cores=NC, num_subcores=NS,
    )
    run = pl.kernel(
        _sc_body,
        out_type=jax.ShapeDtypeStruct((L, E // 8, B // BT, 8, BT),
                                      jnp.float32),
        mesh=mesh,
        scratch_types=[
            pltpu.VMEM((L, BT), jnp.int32),
            pltpu.VMEM((BT, E), jnp.float32),
            pltpu.VMEM((BT, E), jnp.float32),
            pltpu.VMEM((E, BT), jnp.float32),
            pltpu.VMEM((E, BT), jnp.float32),
            pltpu.SemaphoreType.DMA,
            pltpu.SemaphoreType.DMA,
            pltpu.SemaphoreType.DMA,
            pltpu.SemaphoreType.DMA,
        ],
        compiler_params=pltpu.CompilerParams(use_tc_tiling_on_sc=False,
                                             needs_layout_passes=False),
    )
    ids = (old_token_tensor, new_token_tensor, action_tensor, nl_tensor)
    tables = (code_table, code_table, action_table, nl_table)
    outs = []
    for t, tab in zip(ids, tables):
        o5 = run(t.astype(jnp.int32).T, tab)
        outs.append(o5.transpose(2, 4, 0, 1, 3).reshape(B, L, E))
    return tuple(outs)


# speedup vs baseline: 2.0528x; 2.0528x over previous
import jax
import jax.numpy as jnp
from jax import lax
from jax.experimental import pallas as pl
from jax.experimental.pallas import tpu as pltpu
from jax.experimental.pallas import tpu_sc as plsc

NC, NS = 2, 16
NW = NC * NS
BT = 128
NBUF = 2


def _sc_body(idx, table, out, idx_v, gbuf0, gbuf1, tbuf0, tbuf1,
             gsem0, gsem1, ssem0, ssem1):
    wid = lax.axis_index("s") * NC + lax.axis_index("c")
    L, B = idx.shape
    E = table.shape[1]
    b0 = wid * BT
    gbufs = (gbuf0, gbuf1)
    tbufs = (tbuf0, tbuf1)
    gsems = (gsem0, gsem1)
    ssems = (ssem0, ssem1)

    pltpu.sync_copy(idx.at[:, pl.ds(b0, BT)], idx_v)

    row16 = [jnp.arange(16, dtype=jnp.int32) + 16 * k for k in range(BT // 16)]

    def fire_gather(l, b):
        pltpu.async_copy(table.at[idx_v.at[l]], gbufs[b], gsems[b])

    def wait_gather(b):
        pltpu.make_async_copy(table.at[pl.ds(0, BT)], gbufs[b],
                              gsems[b]).wait()

    def wait_stores(b):
        for _ in range(E // 8):
            pltpu.make_async_copy(tbufs[b].at[pl.ds(0, 8)], out.at[0, 0, 0],
                                  ssems[b]).wait()

    def transpose_and_store(l, b):
        gb, tb = gbufs[b], tbufs[b]

        @plsc.parallel_loop(0, E, unroll=8)
        def _(e):
            col = jnp.full((16,), e, dtype=jnp.int32)
            for k in range(BT // 16):
                tb[e, pl.ds(16 * k, 16)] = plsc.load_gather(
                    gb, [row16[k], col])
        for et in range(E // 8):
            pltpu.async_copy(tb.at[pl.ds(8 * et, 8)], out.at[l, et, wid],
                             ssems[b])

    for b in range(NBUF):
        fire_gather(b, b)

    def body(i, carry):
        for b in range(NBUF):
            l = i * NBUF + b
            wait_gather(b)

            @pl.when(l >= NBUF)
            def _():
                wait_stores(b)

            transpose_and_store(l, b)

            @pl.when(l + NBUF < L)
            def _():
                fire_gather(l + NBUF, b)

        return carry

    lax.fori_loop(0, L // NBUF, body, 0)
    for b in range(NBUF):
        wait_stores(b)


def kernel(old_token_tensor, new_token_tensor, action_tensor, nl_tensor,
           code_table, action_table, nl_table):
    B, L = old_token_tensor.shape
    E = code_table.shape[1]
    mesh = plsc.VectorSubcoreMesh(
        core_axis_name="c", subcore_axis_name="s",
        num_cores=NC, num_subcores=NS,
    )
    run = pl.kernel(
        _sc_body,
        out_type=jax.ShapeDtypeStruct((L, E // 8, B // BT, 8, BT),
                                      jnp.float32),
        mesh=mesh,
        scratch_types=[
            pltpu.VMEM((L, BT), jnp.int32),
            pltpu.VMEM((BT, E), jnp.float32),
            pltpu.VMEM((BT, E), jnp.float32),
            pltpu.VMEM((E, BT), jnp.float32),
            pltpu.VMEM((E, BT), jnp.float32),
            pltpu.SemaphoreType.DMA,
            pltpu.SemaphoreType.DMA,
            pltpu.SemaphoreType.DMA,
            pltpu.SemaphoreType.DMA,
        ],
        compiler_params=pltpu.CompilerParams(use_tc_tiling_on_sc=False,
                                             needs_layout_passes=False),
    )
    ids = (old_token_tensor, new_token_tensor, action_tensor, nl_tensor)
    tables = (code_table, code_table, action_table, nl_table)
    outs = []
    for t, tab in zip(ids, tables):
        o5 = run(t.astype(jnp.int32).T, tab)
        outs.append(o5.transpose(2, 4, 0, 1, 3).reshape(B, L, E))
    return tuple(outs)


# speedup vs baseline: 2.0601x; 1.0035x over previous
import jax
import jax.numpy as jnp
from jax import lax
from jax.experimental import pallas as pl
from jax.experimental.pallas import tpu as pltpu
from jax.experimental.pallas import tpu_sc as plsc

NC, NS = 2, 16
NW = NC * NS
BT = 128
NBUF = 2


def _sc_body(idx, table, out, idx_v, gbuf0, gbuf1, tbuf0, tbuf1,
             gsem0, gsem1, ssem0, ssem1):
    wid = lax.axis_index("s") * NC + lax.axis_index("c")
    L, B = idx.shape
    E = table.shape[1]
    b0 = wid * BT
    gbufs = (gbuf0, gbuf1)
    tbufs = (tbuf0, tbuf1)
    gsems = (gsem0, gsem1)
    ssems = (ssem0, ssem1)

    pltpu.sync_copy(idx.at[:, pl.ds(b0, BT)], idx_v)

    row16 = [jnp.arange(16, dtype=jnp.int32) + 16 * k for k in range(BT // 16)]

    def fire_gather(l, b):
        pltpu.async_copy(table.at[idx_v.at[l]], gbufs[b], gsems[b])

    def wait_gather(b):
        pltpu.make_async_copy(table.at[pl.ds(0, BT)], gbufs[b],
                              gsems[b]).wait()

    def wait_stores(b):
        for _ in range(E // 8):
            pltpu.make_async_copy(tbufs[b].at[pl.ds(0, 8)], out.at[0, 0, 0],
                                  ssems[b]).wait()

    def transpose_and_store(l, b):
        gb, tb = gbufs[b], tbufs[b]

        @plsc.parallel_loop(0, E, unroll=16)
        def _(e):
            col = jnp.full((16,), e, dtype=jnp.int32)
            for k in range(BT // 16):
                tb[e, pl.ds(16 * k, 16)] = plsc.load_gather(
                    gb, [row16[k], col])
        for et in range(E // 8):
            pltpu.async_copy(tb.at[pl.ds(8 * et, 8)], out.at[l, et, wid],
                             ssems[b])

    for b in range(NBUF):
        fire_gather(b, b)

    def body(i, carry):
        for b in range(NBUF):
            l = i * NBUF + b
            wait_gather(b)

            @pl.when(l >= NBUF)
            def _():
                wait_stores(b)

            transpose_and_store(l, b)

            @pl.when(l + NBUF < L)
            def _():
                fire_gather(l + NBUF, b)

        return carry

    lax.fori_loop(0, L // NBUF, body, 0)
    for b in range(NBUF):
        wait_stores(b)


def kernel(old_token_tensor, new_token_tensor, action_tensor, nl_tensor,
           code_table, action_table, nl_table):
    B, L = old_token_tensor.shape
    E = code_table.shape[1]
    mesh = plsc.VectorSubcoreMesh(
        core_axis_name="c", subcore_axis_name="s",
        num_cores=NC, num_subcores=NS,
    )
    run = pl.kernel(
        _sc_body,
        out_type=jax.ShapeDtypeStruct((L, E // 8, B // BT, 8, BT),
                                      jnp.float32),
        mesh=mesh,
        scratch_types=[
            pltpu.VMEM((L, BT), jnp.int32),
            pltpu.VMEM((BT, E), jnp.float32),
            pltpu.VMEM((BT, E), jnp.float32),
            pltpu.VMEM((E, BT), jnp.float32),
            pltpu.VMEM((E, BT), jnp.float32),
            pltpu.SemaphoreType.DMA,
            pltpu.SemaphoreType.DMA,
            pltpu.SemaphoreType.DMA,
            pltpu.SemaphoreType.DMA,
        ],
        compiler_params=pltpu.CompilerParams(use_tc_tiling_on_sc=False,
                                             needs_layout_passes=False),
    )
    ids = (old_token_tensor, new_token_tensor, action_tensor, nl_tensor)
    tables = (code_table, code_table, action_table, nl_table)
    outs = []
    for t, tab in zip(ids, tables):
        o5 = run(t.astype(jnp.int32).T, tab)
        outs.append(o5.transpose(2, 4, 0, 1, 3).reshape(B, L, E))
    return tuple(outs)


# speedup vs baseline: 6.2112x; 3.0150x over previous
import jax
import jax.numpy as jnp
from jax import lax
from jax.experimental import pallas as pl
from jax.experimental.pallas import tpu as pltpu
from jax.experimental.pallas import tpu_sc as plsc

NC, NS = 2, 16
NW = NC * NS
BT = 128
NBUF = 2


def _sc_body(idx, table, out, idx_v, gbuf0, gbuf1, tbuf0, tbuf1,
             gsem0, gsem1, ssem0, ssem1):
    wid = lax.axis_index("s") * NC + lax.axis_index("c")
    L, B = idx.shape
    E = table.shape[1]
    b0 = wid * BT
    gbufs = (gbuf0, gbuf1)
    tbufs = (tbuf0, tbuf1)
    gsems = (gsem0, gsem1)
    ssems = (ssem0, ssem1)

    pltpu.sync_copy(idx.at[:, pl.ds(b0, BT)], idx_v)

    row16 = [jnp.arange(16, dtype=jnp.int32) + 16 * k for k in range(BT // 16)]

    def fire_gather(l, b):
        pltpu.async_copy(table.at[idx_v.at[l]], gbufs[b], gsems[b])

    def wait_gather(b):
        pltpu.make_async_copy(table.at[pl.ds(0, BT)], gbufs[b],
                              gsems[b]).wait()

    def wait_stores(b):
        for _ in range(E // 8):
            pltpu.make_async_copy(tbufs[b].at[pl.ds(0, 8)], out.at[0, 0, 0],
                                  ssems[b]).wait()

    def transpose_and_store(l, b):
        gb, tb = gbufs[b], tbufs[b]
        lane = row16[0]

        @plsc.parallel_loop(0, E, unroll=16)
        def _(e):
            col = (e + lane) & (E - 1)
            for k in range(BT // 16):
                v = plsc.load_gather(gb, [row16[k], col])
                plsc.store_scatter(tb, [col, row16[k]], v)
        for et in range(E // 8):
            pltpu.async_copy(tb.at[pl.ds(8 * et, 8)], out.at[l, et, wid],
                             ssems[b])

    for b in range(NBUF):
        fire_gather(b, b)

    def body(i, carry):
        for b in range(NBUF):
            l = i * NBUF + b
            wait_gather(b)

            @pl.when(l >= NBUF)
            def _():
                wait_stores(b)

            transpose_and_store(l, b)

            @pl.when(l + NBUF < L)
            def _():
                fire_gather(l + NBUF, b)

        return carry

    lax.fori_loop(0, L // NBUF, body, 0)
    for b in range(NBUF):
        wait_stores(b)


def kernel(old_token_tensor, new_token_tensor, action_tensor, nl_tensor,
           code_table, action_table, nl_table):
    B, L = old_token_tensor.shape
    E = code_table.shape[1]
    mesh = plsc.VectorSubcoreMesh(
        core_axis_name="c", subcore_axis_name="s",
        num_cores=NC, num_subcores=NS,
    )
    run = pl.kernel(
        _sc_body,
        out_type=jax.ShapeDtypeStruct((L, E // 8, B // BT, 8, BT),
                                      jnp.float32),
        mesh=mesh,
        scratch_types=[
            pltpu.VMEM((L, BT), jnp.int32),
            pltpu.VMEM((BT, E), jnp.float32),
            pltpu.VMEM((BT, E), jnp.float32),
            pltpu.VMEM((E, BT), jnp.float32),
            pltpu.VMEM((E, BT), jnp.float32),
            pltpu.SemaphoreType.DMA,
            pltpu.SemaphoreType.DMA,
            pltpu.SemaphoreType.DMA,
            pltpu.SemaphoreType.DMA,
        ],
        compiler_params=pltpu.CompilerParams(use_tc_tiling_on_sc=False,
                                             needs_layout_passes=False),
    )
    ids = (old_token_tensor, new_token_tensor, action_tensor, nl_tensor)
    tables = (code_table, code_table, action_table, nl_table)
    outs = []
    for t, tab in zip(ids, tables):
        o5 = run(t.astype(jnp.int32).T, tab)
        outs.append(o5.transpose(2, 4, 0, 1, 3).reshape(B, L, E))
    return tuple(outs)
